# Initial kernel scaffold; baseline (speedup 1.0000x reference)
import jax
import jax.numpy as jnp
from jax.experimental import pallas as pl


def kernel(x, edge_index, enc_W, enc_b, W1, att_src1, att_dst1, b1, W2, att_src2, att_dst2, b2, W3, att_src3, att_dst3, b3, dec_W, dec_b):
    raise NotImplementedError("write your pallas kernel here")


# speedup vs baseline: 11.1217x; 11.1217x over previous
import functools

import jax
import jax.numpy as jnp
from jax import lax
from jax.experimental import pallas as pl
from jax.experimental.pallas import tpu as pltpu
from jax.experimental.pallas import tpu_sc as plsc

N = 10000
E = 160000
D = 128
H4 = 4

NC = 2
NS = 16
L = 16

CH = 128
E_PAD = 163840
N_ACC = 10240
DUMMY = N_ACC - 8
ROWS_PT = N_ACC // NS

BN = 400
GRID = N // BN


def _make_sc_layer(heads):
    hpc = 2 if heads == 4 else 1
    epc = E_PAD if heads == 4 else E_PAD // NC
    ept = epc // NS
    nch = ept // CH
    nmaj = heads if heads == 4 else NC

    mesh = plsc.VectorSubcoreMesh(core_axis_name="c", subcore_axis_name="s")

    @functools.partial(
        pl.kernel,
        out_type=[
            jax.ShapeDtypeStruct((nmaj * N_ACC, D), jnp.float32),
            jax.ShapeDtypeStruct((nmaj * N_ACC,), jnp.float32),
        ],
        mesh=mesh,
        compiler_params=pltpu.CompilerParams(needs_layout_passes=False),
        scratch_types=[
            pltpu.VMEM((N,), jnp.float32),
            pltpu.VMEM((N,), jnp.float32),
            pltpu.VMEM((heads, L), jnp.float32),
            pltpu.VMEM((CH,), jnp.int32),
            pltpu.VMEM((CH,), jnp.int32),
            pltpu.VMEM((CH,), jnp.int32),
            pltpu.VMEM((CH,), jnp.float32),
            pltpu.VMEM((CH,), jnp.float32),
            pltpu.VMEM((CH, D), jnp.float32),
            pltpu.VMEM((ROWS_PT,), jnp.float32),
            pltpu.VMEM_SHARED((N_ACC, D), jnp.float32),
            pltpu.VMEM_SHARED((N_ACC,), jnp.float32),
            pltpu.VMEM_SHARED((N_ACC,), jnp.float32),
            pltpu.SemaphoreType.DMA,
        ],
    )
    def sc_layer(h_hbm, as_hbm, ad_hbm, cb_hbm, src_hbm, dst_hbm,
                 num_hbm, den_hbm,
                 as_v, ad_v, cb_v, srcb, dstb, idxb, exa, exb, rows,
                 z1d, accum, den0, den1, dsem):
        cid = lax.axis_index("c")
        sid = lax.axis_index("s")
        dens = [den0, den1][:hpc]

        pltpu.sync_copy(cb_hbm, cb_v)
        crows = [cb_v[i, :] for i in range(heads)]
        if heads == 4:
            cvs = [jnp.where(cid == 0, crows[j], crows[2 + j]) for j in range(hpc)]
        else:
            cvs = [crows[0]]

        def stage_tables(j):
            if heads == 4:
                toff = (cid * hpc + j) * N
            else:
                toff = 0
            pltpu.sync_copy(as_hbm.at[pl.ds(toff, N)], as_v)
            pltpu.sync_copy(ad_hbm.at[pl.ds(toff, N)], ad_v)

        zv = jnp.zeros((L,), jnp.float32)
        for i in range(ROWS_PT // L):
            z1d[pl.ds(i * L, L)] = zv
        for j in range(hpc):
            pltpu.sync_copy(z1d, dens[j].at[pl.ds(sid * ROWS_PT, ROWS_PT)])
        plsc.subcore_barrier()

        base0 = sid * ept if heads == 4 else cid * epc + sid * ept

        for j in range(hpc):
            stage_tables(j)

            def phase_a(k, carry):
                base_g = base0 + k * CH
                pltpu.sync_copy(src_hbm.at[pl.ds(base_g, CH)], srcb)
                pltpu.sync_copy(dst_hbm.at[pl.ds(base_g, CH)], dstb)
                for t in range(CH // L):
                    sv = srcb[pl.ds(t * L, L)]
                    dv = jnp.minimum(dstb[pl.ds(t * L, L)], jnp.int32(N - 1))
                    av = plsc.load_gather(as_v, [sv])
                    bv = plsc.load_gather(ad_v, [dv])
                    tt = av + bv
                    ee = jnp.maximum(tt, 0.2 * tt)
                    exa[pl.ds(t * L, L)] = jnp.exp(ee - cvs[j])
                pltpu.sync_copy(exa, dens[j].at[dstb], add=True)
                return carry

            lax.fori_loop(0, nch, phase_a, 0)
        plsc.subcore_barrier()

        for j in range(hpc):
            off = (cid * hpc + j) * N_ACC + sid * ROWS_PT
            pltpu.sync_copy(dens[j].at[pl.ds(sid * ROWS_PT, ROWS_PT)],
                            den_hbm.at[pl.ds(off, ROWS_PT)])

        for j in range(hpc):
            orow = cid * hpc + j
            stage_tables(j)

            def zrows(r, carry):
                for t in range(D // L):
                    rows[r, pl.ds(t * L, L)] = jnp.zeros((L,), jnp.float32)
                return carry

            lax.fori_loop(0, CH, zrows, 0)
            for i in range(ROWS_PT // CH):
                pltpu.sync_copy(rows, accum.at[pl.ds(sid * ROWS_PT + i * CH, CH)])
            plsc.subcore_barrier()

            def phase_b(k, carry):
                base_g = base0 + k * CH
                pltpu.sync_copy(src_hbm.at[pl.ds(base_g, CH)], srcb)
                pltpu.sync_copy(dst_hbm.at[pl.ds(base_g, CH)], dstb)
                if heads == 4:
                    for t in range(CH // L):
                        sv = srcb[pl.ds(t * L, L)]
                        idxb[pl.ds(t * L, L)] = sv * H4 + orow
                    pltpu.async_copy(h_hbm.at[idxb], rows, dsem).wait()
                else:
                    pltpu.async_copy(h_hbm.at[srcb], rows, dsem).wait()
                for t in range(CH // L):
                    sv = srcb[pl.ds(t * L, L)]
                    dv = jnp.minimum(dstb[pl.ds(t * L, L)], jnp.int32(N - 1))
                    av = plsc.load_gather(as_v, [sv])
                    bv = plsc.load_gather(ad_v, [dv])
                    tt = av + bv
                    ee = jnp.maximum(tt, 0.2 * tt)
                    exb[pl.ds(t * L, L)] = jnp.exp(ee - cvs[j])

                def scale(e, c2):
                    ev = plsc.load_gather(exb, [jnp.full((L,), e, jnp.int32)])
                    for t in range(D // L):
                        rows[e, pl.ds(t * L, L)] = rows[e, pl.ds(t * L, L)] * ev
                    return c2

                lax.fori_loop(0, CH, scale, 0)
                pltpu.sync_copy(rows, accum.at[dstb], add=True)
                return carry

            lax.fori_loop(0, nch, phase_b, 0)
            plsc.subcore_barrier()

            for i in range(ROWS_PT // CH):
                r = sid * ROWS_PT + i * CH
                off = orow * N_ACC + r
                pltpu.sync_copy(accum.at[pl.ds(r, CH)],
                                num_hbm.at[pl.ds(off, CH)])
            plsc.subcore_barrier()

    return sc_layer


@functools.lru_cache(maxsize=None)
def _sc_layer(heads):
    return _make_sc_layer(heads)


def _enc_body(x_ref, ew_ref, eb_ref, w1_ref, s1_ref, d1_ref,
              h1_ref, as_ref, ad_ref, pmax_ref):
    h0 = jnp.dot(x_ref[...], ew_ref[...], preferred_element_type=jnp.float32)
    h0 = h0 + eb_ref[...]
    h1 = jnp.dot(h0, w1_ref[...], preferred_element_type=jnp.float32)
    hr = h1.reshape(BN, H4, D)
    asb = jnp.sum(hr * s1_ref[...][None], axis=-1)
    adb = jnp.sum(hr * d1_ref[...][None], axis=-1)
    h1_ref[...] = h1
    as_ref[...] = asb
    ad_ref[...] = adb
    pmax_ref[...] = jnp.stack([jnp.max(asb, 0), jnp.max(adb, 0)])[None]


def _full(shape):
    return pl.BlockSpec(shape, lambda i: (0,) * len(shape))


def _enc_call(x, enc_W, enc_b, W1, s1, d1):
    return pl.pallas_call(
        _enc_body,
        grid=(GRID,),
        in_specs=[
            pl.BlockSpec((BN, D), lambda i: (i, 0)),
            _full((D, D)), _full((1, D)), _full((D, H4 * D)),
            _full((H4, D)), _full((H4, D)),
        ],
        out_specs=[
            pl.BlockSpec((BN, H4 * D), lambda i: (i, 0)),
            pl.BlockSpec((BN, H4), lambda i: (i, 0)),
            pl.BlockSpec((BN, H4), lambda i: (i, 0)),
            pl.BlockSpec((1, 2, H4), lambda i: (i, 0, 0)),
        ],
        out_shape=[
            jax.ShapeDtypeStruct((N, H4 * D), jnp.float32),
            jax.ShapeDtypeStruct((N, H4), jnp.float32),
            jax.ShapeDtypeStruct((N, H4), jnp.float32),
            jax.ShapeDtypeStruct((GRID, 2, H4), jnp.float32),
        ],
    )(x, enc_W, enc_b, W1, s1, d1)


def _make_mid(dn, hn):
    def body(num_ref, den_ref, as_ref, ad_ref, c_ref, h_ref, b_ref,
             wn_ref, sn_ref, dnr_ref,
             hn_ref, asn_ref, adn_ref, pmax_ref):
        den = den_ref[...]
        asb = as_ref[...]
        adb = ad_ref[...]
        ts = asb + adb
        es = jnp.maximum(ts, 0.2 * ts)
        exs = jnp.exp(es - c_ref[...])
        hc = h_ref[...]
        dent = den + exs
        outs = []
        for hh in range(H4):
            numer = (num_ref[hh] + hc[:, hh * D:(hh + 1) * D]
                     * exs[:, hh:hh + 1])
            outs.append(numer / dent[:, hh:hh + 1])
        xn = jnp.concatenate(outs, axis=1) + b_ref[...]
        xn = jnp.maximum(xn, 0.0)
        hnx = jnp.dot(xn, wn_ref[...], preferred_element_type=jnp.float32)
        hr = hnx.reshape(BN, hn, D)
        asn = jnp.sum(hr * sn_ref[...][None], axis=-1)
        adn = jnp.sum(hr * dnr_ref[...][None], axis=-1)
        hn_ref[...] = hnx
        asn_ref[...] = asn
        adn_ref[...] = adn
        pmax_ref[...] = jnp.stack([jnp.max(asn, 0), jnp.max(adn, 0)])[None]

    def call(num, den_nt, asb, adb, c, h, b, wn, sn, dnr):
        return pl.pallas_call(
            body,
            grid=(GRID,),
            in_specs=[
                pl.BlockSpec((H4, BN, D), lambda i: (0, i, 0)),
                pl.BlockSpec((BN, H4), lambda i: (i, 0)),
                pl.BlockSpec((BN, H4), lambda i: (i, 0)),
                pl.BlockSpec((BN, H4), lambda i: (i, 0)),
                _full((1, H4)),
                pl.BlockSpec((BN, H4 * D), lambda i: (i, 0)),
                _full((1, H4 * D)),
                _full((H4 * D, dn)),
                _full((hn, D)), _full((hn, D)),
            ],
            out_specs=[
                pl.BlockSpec((BN, dn), lambda i: (i, 0)),
                pl.BlockSpec((BN, hn), lambda i: (i, 0)),
                pl.BlockSpec((BN, hn), lambda i: (i, 0)),
                pl.BlockSpec((1, 2, hn), lambda i: (i, 0, 0)),
            ],
            out_shape=[
                jax.ShapeDtypeStruct((N, dn), jnp.float32),
                jax.ShapeDtypeStruct((N, hn), jnp.float32),
                jax.ShapeDtypeStruct((N, hn), jnp.float32),
                jax.ShapeDtypeStruct((GRID, 2, hn), jnp.float32),
            ],
        )(num, den_nt, asb, adb, c, h, b, wn, sn, dnr)

    return call


def _final_body(num_ref, den_ref, as_ref, ad_ref, c_ref, h_ref, b_ref,
                dw_ref, db_ref, y_ref):
    num = num_ref[0] + num_ref[1]
    den = den_ref[...]
    dsum = den[:, 0] + den[:, 1]
    ts = as_ref[...] + ad_ref[...]
    es = jnp.maximum(ts, 0.2 * ts)
    exs = jnp.exp(es - c_ref[...])
    numer = num + h_ref[...] * exs
    dent = dsum + exs[:, 0]
    out = numer / dent[:, None] + b_ref[...]
    y = jnp.dot(out, dw_ref[...], preferred_element_type=jnp.float32)
    y_ref[...] = y + db_ref[...]


def _final_call(num, den_nt, asb, adb, c, h, b, dw, db):
    return pl.pallas_call(
        _final_body,
        grid=(GRID,),
        in_specs=[
            pl.BlockSpec((NC, BN, D), lambda i: (0, i, 0)),
            pl.BlockSpec((BN, NC), lambda i: (i, 0)),
            pl.BlockSpec((BN, 1), lambda i: (i, 0)),
            pl.BlockSpec((BN, 1), lambda i: (i, 0)),
            _full((1, 1)),
            pl.BlockSpec((BN, D), lambda i: (i, 0)),
            _full((1, D)),
            _full((D, D)), _full((1, D)),
        ],
        out_specs=pl.BlockSpec((BN, D), lambda i: (i, 0)),
        out_shape=jax.ShapeDtypeStruct((N, D), jnp.float32),
    )(num, den_nt, asb, adb, c, h, b, dw, db)


_mid_call_44 = _make_mid(H4 * D, H4)
_mid_call_41 = _make_mid(D, 1)


def _cmax(pmax):
    return jnp.max(pmax[:, 0], axis=0) + jnp.max(pmax[:, 1], axis=0)


def kernel(x, edge_index, enc_W, enc_b, W1, att_src1, att_dst1, b1,
           W2, att_src2, att_dst2, b2, W3, att_src3, att_dst3, b3,
           dec_W, dec_b):
    srcp = jnp.concatenate(
        [edge_index[0], jnp.zeros((E_PAD - E,), jnp.int32)])
    dstp = jnp.concatenate(
        [edge_index[1], jnp.full((E_PAD - E,), DUMMY, jnp.int32)])

    s1 = att_src1.reshape(H4, D)
    d1 = att_dst1.reshape(H4, D)
    s2 = att_src2.reshape(H4, D)
    d2 = att_dst2.reshape(H4, D)
    s3 = att_src3.reshape(1, D)
    d3 = att_dst3.reshape(1, D)

    h1, as1, ad1, pmax1 = _enc_call(x, enc_W, enc_b.reshape(1, D), W1, s1, d1)
    c1 = _cmax(pmax1)
    cb1 = jnp.broadcast_to(c1[:, None], (H4, L))
    num1, den1 = _sc_layer(4)(h1.reshape(N * H4, D),
                      as1.T.reshape(-1), ad1.T.reshape(-1), cb1, srcp, dstp)
    num1 = num1.reshape(H4, N_ACC, D)
    den1 = den1.reshape(H4, N_ACC)[:, :N].T

    h2, as2, ad2, pmax2 = _mid_call_44(
        num1, den1, as1, ad1, c1[None], h1, b1.reshape(1, H4 * D), W2, s2, d2)
    c2 = _cmax(pmax2)
    cb2 = jnp.broadcast_to(c2[:, None], (H4, L))
    num2, den2 = _sc_layer(4)(h2.reshape(N * H4, D),
                      as2.T.reshape(-1), ad2.T.reshape(-1), cb2, srcp, dstp)
    num2 = num2.reshape(H4, N_ACC, D)
    den2 = den2.reshape(H4, N_ACC)[:, :N].T

    h3, as3, ad3, pmax3 = _mid_call_41(
        num2, den2, as2, ad2, c2[None], h2, b2.reshape(1, H4 * D), W3, s3, d3)
    c3 = _cmax(pmax3)
    cb3 = jnp.broadcast_to(c3[:, None], (1, L))
    num3, den3 = _sc_layer(1)(h3, as3.T.reshape(-1), ad3.T.reshape(-1), cb3, srcp, dstp)
    num3 = num3.reshape(NC, N_ACC, D)
    den3 = den3.reshape(NC, N_ACC)[:, :N].T

    return _final_call(num3, den3, as3, ad3, c3[None], h3,
                       b3.reshape(1, D), dec_W, dec_b.reshape(1, D))
